# Initial kernel scaffold; baseline (speedup 1.0000x reference)
import jax
import jax.numpy as jnp
from jax.experimental import pallas as pl


def kernel(X_sites, X_NSs, W, b):
    raise NotImplementedError("write your pallas kernel here")


# speedup vs baseline: 4.4925x; 4.4925x over previous
import functools

import jax
import jax.numpy as jnp
from jax import lax
from jax.experimental import pallas as pl
from jax.experimental.pallas import tpu as pltpu
from jax.experimental.pallas import tpu_sc as plsc

N_SITES = 10000
D_FEAT = 128
N_PERM = 8
NBR = 6
OUT_FEAT = 32
PROJ = NBR * OUT_FEAT

NW = 32
SITES_PER_W = 320
N_PAD = NW * SITES_PER_W
BLK_SITES = 64
NBLK = SITES_PER_W // BLK_SITES
IDX_PER_BLK = BLK_SITES * N_PERM * NBR
IDX_ROWS_PER_BLK = IDX_PER_BLK // 128
PAIRS_PER_BLK = BLK_SITES * N_PERM
IDX_ROWS_PER_W = NBLK * IDX_ROWS_PER_BLK


def _proj_body(x_ref, w_ref, y_ref):
    y_ref[...] = lax.dot_general(
        x_ref[...], w_ref[...], (((1,), (1,)), ((), ())),
        preferred_element_type=jnp.float32)


def _project(x_sites, w_big):
    return pl.pallas_call(
        _proj_body,
        out_shape=jax.ShapeDtypeStruct((N_SITES, PROJ), jnp.float32),
    )(x_sites, w_big)


_sc_mesh = plsc.VectorSubcoreMesh(core_axis_name="c", subcore_axis_name="s")


@functools.partial(
    pl.kernel,
    mesh=_sc_mesh,
    compiler_params=pltpu.CompilerParams(use_tc_tiling_on_sc=False),
    out_type=jax.ShapeDtypeStruct((N_PAD * N_PERM, OUT_FEAT), jnp.float32),
    scratch_types=[
        pltpu.VMEM((IDX_ROWS_PER_BLK, 128), jnp.int32),
        pltpu.VMEM((IDX_PER_BLK, OUT_FEAT), jnp.float32),
        pltpu.VMEM((PAIRS_PER_BLK, OUT_FEAT), jnp.float32),
        pltpu.SemaphoreType.DMA,
    ],
)
def _gather_sum(y_hbm, idx_hbm, x1_hbm, idx_v, rows_v, x1_v, sem):
    wid = lax.axis_index("s") * 2 + lax.axis_index("c")

    def blk_body(bi, carry):
        row0 = wid * IDX_ROWS_PER_W + bi * IDX_ROWS_PER_BLK
        pltpu.sync_copy(idx_hbm.at[pl.ds(row0, IDX_ROWS_PER_BLK)], idx_v)
        copies = []
        for j in range(IDX_ROWS_PER_BLK):
            copies.append(pltpu.async_copy(
                y_hbm.at[idx_v.at[j]],
                rows_v.at[pl.ds(j * 128, 128)], sem))
        for c in copies:
            c.wait()

        def pair_body(q, c2):
            r0 = q * NBR
            for h in range(OUT_FEAT // 16):
                acc = rows_v[r0, pl.ds(h * 16, 16)]
                for j in range(1, NBR):
                    acc = acc + rows_v[r0 + j, pl.ds(h * 16, 16)]
                x1_v[q, pl.ds(h * 16, 16)] = acc
            return c2

        lax.fori_loop(0, PAIRS_PER_BLK, pair_body, 0, unroll=2)
        out0 = (wid * SITES_PER_W + bi * BLK_SITES) * N_PERM
        pltpu.sync_copy(x1_v, x1_hbm.at[pl.ds(out0, PAIRS_PER_BLK)])
        return carry

    lax.fori_loop(0, NBLK, blk_body, 0)


_BLK_C = 1024
_LOG2 = 0.6931471805599453


def _act_body(x1_ref, b_ref, o_ref):
    x = x1_ref[...]
    b = b_ref[...]
    acc = jnp.zeros((_BLK_C, OUT_FEAT), jnp.float32)
    for p in range(N_PERM):
        z = x[:, p * OUT_FEAT:(p + 1) * OUT_FEAT] + b
        acc = acc + jax.nn.softplus(z)
    o_ref[...] = acc - jnp.float32(N_PERM * _LOG2)


def _activate(x1, b2):
    grid = (N_PAD // _BLK_C,)
    return pl.pallas_call(
        _act_body,
        grid=grid,
        in_specs=[
            pl.BlockSpec((_BLK_C, N_PERM * OUT_FEAT), lambda i: (i, 0)),
            pl.BlockSpec((1, OUT_FEAT), lambda i: (0, 0)),
        ],
        out_specs=pl.BlockSpec((_BLK_C, OUT_FEAT), lambda i: (i, 0)),
        out_shape=jax.ShapeDtypeStruct((N_PAD, OUT_FEAT), jnp.float32),
    )(x1, b2)


def kernel(X_sites, X_NSs, W, b):
    w_big = W.reshape(OUT_FEAT, NBR, D_FEAT).transpose(1, 0, 2).reshape(PROJ, D_FEAT)

    y = _project(X_sites, w_big)
    table = y.reshape(N_SITES * NBR, OUT_FEAT)

    idx = X_NSs.astype(jnp.int32) * NBR + jnp.arange(NBR, dtype=jnp.int32)
    flat = idx.reshape(-1)
    flat = jnp.pad(flat, (0, N_PAD * N_PERM * NBR - flat.shape[0]))
    idx_arr = flat.reshape(N_PAD * N_PERM * NBR // 128, 128)

    x1 = _gather_sum(table, idx_arr)

    out = _activate(x1.reshape(N_PAD, N_PERM * OUT_FEAT), b.reshape(1, OUT_FEAT))
    return out[:N_SITES]


# speedup vs baseline: 5.9022x; 1.3138x over previous
import functools

import jax
import jax.numpy as jnp
from jax import lax
from jax.experimental import pallas as pl
from jax.experimental.pallas import tpu as pltpu
from jax.experimental.pallas import tpu_sc as plsc

N_SITES = 10000
D_FEAT = 128
N_PERM = 8
NBR = 6
OUT_FEAT = 32
PROJ = NBR * OUT_FEAT

NW = 32
SITES_PER_W = 320
N_PAD = NW * SITES_PER_W
PAIRS_PER_W = SITES_PER_W * N_PERM
N_PAIRS = N_PAD * N_PERM


def _proj_body(x_ref, w_ref, y_ref):
    y_ref[...] = lax.dot_general(
        x_ref[...], w_ref[...], (((1,), (1,)), ((), ())),
        preferred_element_type=jnp.float32)


def _project(x_sites, w_big):
    return pl.pallas_call(
        _proj_body,
        out_shape=jax.ShapeDtypeStruct((N_SITES, PROJ), jnp.float32),
    )(x_sites, w_big)


_sc_mesh = plsc.VectorSubcoreMesh(core_axis_name="c", subcore_axis_name="s")


@functools.partial(
    pl.kernel,
    mesh=_sc_mesh,
    compiler_params=pltpu.CompilerParams(use_tc_tiling_on_sc=False),
    out_type=jax.ShapeDtypeStruct((N_PAIRS, OUT_FEAT), jnp.float32),
    scratch_types=[
        pltpu.VMEM((NBR, PAIRS_PER_W), jnp.int32),
        pltpu.VMEM((PAIRS_PER_W, OUT_FEAT), jnp.float32),
        pltpu.SemaphoreType.DMA,
        pltpu.SemaphoreType.DMA,
    ],
)
def _gather_sum(y_hbm, idx_hbm, x1_hbm, idx_v, x1_v, sem0, sem):
    wid = lax.axis_index("s") * 2 + lax.axis_index("c")
    p0 = wid * PAIRS_PER_W
    for j in range(NBR):
        pltpu.sync_copy(idx_hbm.at[j, pl.ds(p0, PAIRS_PER_W)], idx_v.at[j])
    pltpu.async_copy(y_hbm.at[idx_v.at[0]], x1_v, sem0).wait()
    copies = [
        pltpu.async_copy(y_hbm.at[idx_v.at[j]], x1_v, sem, add=True)
        for j in range(1, NBR)
    ]
    for c in copies:
        c.wait()
    pltpu.sync_copy(x1_v, x1_hbm.at[pl.ds(p0, PAIRS_PER_W)])


_BLK_C = 1024
_LOG2 = 0.6931471805599453


def _act_body(x1_ref, b_ref, o_ref):
    x = x1_ref[...]
    b = b_ref[...]
    acc = jnp.zeros((_BLK_C, OUT_FEAT), jnp.float32)
    for p in range(N_PERM):
        z = x[:, p * OUT_FEAT:(p + 1) * OUT_FEAT] + b
        acc = acc + jax.nn.softplus(z)
    o_ref[...] = acc - jnp.float32(N_PERM * _LOG2)


def _activate(x1, b2):
    grid = (N_PAD // _BLK_C,)
    return pl.pallas_call(
        _act_body,
        grid=grid,
        in_specs=[
            pl.BlockSpec((_BLK_C, N_PERM * OUT_FEAT), lambda i: (i, 0)),
            pl.BlockSpec((1, OUT_FEAT), lambda i: (0, 0)),
        ],
        out_specs=pl.BlockSpec((_BLK_C, OUT_FEAT), lambda i: (i, 0)),
        out_shape=jax.ShapeDtypeStruct((N_PAD, OUT_FEAT), jnp.float32),
    )(x1, b2)


def kernel(X_sites, X_NSs, W, b):
    w_big = W.reshape(OUT_FEAT, NBR, D_FEAT).transpose(1, 0, 2).reshape(PROJ, D_FEAT)

    y = _project(X_sites, w_big)
    table = y.reshape(N_SITES * NBR, OUT_FEAT)

    idx = X_NSs.astype(jnp.int32) * NBR + jnp.arange(NBR, dtype=jnp.int32)
    idx = idx.reshape(N_SITES * N_PERM, NBR)
    idx = jnp.pad(idx, ((0, N_PAIRS - N_SITES * N_PERM), (0, 0)))
    idx_arr = idx.T

    x1 = _gather_sum(table, idx_arr)

    out = _activate(x1.reshape(N_PAD, N_PERM * OUT_FEAT), b.reshape(1, OUT_FEAT))
    return out[:N_SITES]


# speedup vs baseline: 14.5040x; 2.4574x over previous
import functools

import jax
import jax.numpy as jnp
from jax import lax
from jax.experimental import pallas as pl
from jax.experimental.pallas import tpu as pltpu
from jax.experimental.pallas import tpu_sc as plsc

N_SITES = 10000
D_FEAT = 128
N_PERM = 8
NBR = 6
OUT_FEAT = 32
PROJ = NBR * OUT_FEAT

NW = 32
SITES_PER_W = 320
N_PAD = NW * SITES_PER_W
PAIRS_PER_W = SITES_PER_W * N_PERM
N_PAIRS = N_PAD * N_PERM


def _proj_body(x_ref, w_ref, y_ref):
    y_ref[0] = lax.dot_general(
        x_ref[...], w_ref[0], (((1,), (1,)), ((), ())),
        preferred_element_type=jnp.float32).astype(jnp.bfloat16)


def _project(x_sites, w3):
    return pl.pallas_call(
        _proj_body,
        grid=(2,),
        in_specs=[
            pl.BlockSpec((N_SITES, D_FEAT), lambda h: (0, 0)),
            pl.BlockSpec((1, D_FEAT, D_FEAT), lambda h: (h, 0, 0)),
        ],
        out_specs=pl.BlockSpec((1, N_SITES, D_FEAT), lambda h: (h, 0, 0)),
        out_shape=jax.ShapeDtypeStruct((2, N_SITES, D_FEAT), jnp.bfloat16),
    )(x_sites, w3)


_sc_mesh = plsc.VectorSubcoreMesh(core_axis_name="c", subcore_axis_name="s")


TAB_ROWS = 2 * N_SITES * 4
ROWS_PER_TILE = TAB_ROWS // 16
_JOFF = [(j >> 2) * (N_SITES * 4) + (j & 3) for j in range(NBR)]


CHUNK = SITES_PER_W // 2


@functools.partial(
    pl.kernel,
    mesh=_sc_mesh,
    compiler_params=pltpu.CompilerParams(
        use_tc_tiling_on_sc=False, needs_layout_passes=False),
    out_type=jax.ShapeDtypeStruct((N_PAD, N_PERM, OUT_FEAT), jnp.bfloat16),
    scratch_types=[
        pltpu.VMEM_SHARED((TAB_ROWS, OUT_FEAT), jnp.bfloat16),
        pltpu.VMEM((2, NBR, N_PERM * CHUNK), jnp.int32),
        pltpu.VMEM((N_PERM * CHUNK, OUT_FEAT), jnp.bfloat16),
        pltpu.SemaphoreType.DMA,
        pltpu.SemaphoreType.DMA,
        pltpu.SemaphoreType.DMA,
    ],
)
def _gather_sum(y_hbm, nbr_hbm, x1_hbm, tab_s, jl_v, x1_v, semt, sem0, sem):
    sid = lax.axis_index("s")
    wid = sid * 2 + lax.axis_index("c")

    t = pltpu.async_copy(
        y_hbm.at[pl.ds(sid * ROWS_PER_TILE, ROWS_PER_TILE)],
        tab_s.at[pl.ds(sid * ROWS_PER_TILE, ROWS_PER_TILE)], semt)

    idx_copies = []
    for ch in range(2):
        n0 = wid * SITES_PER_W + ch * CHUNK
        for j in range(NBR):
            for p in range(N_PERM):
                idx_copies.append(pltpu.async_copy(
                    nbr_hbm.at[j, p, pl.ds(n0, CHUNK)],
                    jl_v.at[ch, j, pl.ds(p * CHUNK, CHUNK)], sem0))

    def zero(k, carry):
        x1_v[k, :] = jnp.zeros((OUT_FEAT,), jnp.bfloat16)
        return carry

    lax.fori_loop(0, N_PERM * CHUNK, zero, 0, unroll=8)
    for c in idx_copies:
        c.wait()

    def fold(k, carry):
        base = k * 16
        for ch in range(2):
            for j in range(NBR):
                v = jl_v[ch, j, pl.ds(base, 16)]
                jl_v[ch, j, pl.ds(base, 16)] = v * 4 + _JOFF[j]
        return carry

    lax.fori_loop(0, (N_PERM * CHUNK) // 16, fold, 0, unroll=2)

    t.wait()
    plsc.subcore_barrier()

    for ch in range(2):
        n0 = wid * SITES_PER_W + ch * CHUNK
        adds = [
            pltpu.async_copy(tab_s.at[jl_v.at[ch, j]], x1_v, sem, add=True)
            for j in range(NBR)
        ]
        for c in adds:
            c.wait()

        outs = [
            pltpu.async_copy(x1_v.at[pl.ds(p * CHUNK, CHUNK)],
                             x1_hbm.at[pl.ds(n0, CHUNK), p], sem0)
            for p in range(N_PERM)
        ]
        for c in outs:
            c.wait()
        if ch == 0:
            lax.fori_loop(0, N_PERM * CHUNK, zero, 0, unroll=8)


_BLK_C = 1000
_LOG2 = 0.6931471805599453
_LOG2E = 1.4426950408889634


def _act_body(x1_ref, b_ref, o_ref):
    x = x1_ref[...].astype(jnp.float32)
    b = b_ref[...]
    acc = jnp.zeros((_BLK_C, OUT_FEAT), jnp.float32)
    for p in range(N_PERM):
        z = x[:, p * OUT_FEAT:(p + 1) * OUT_FEAT] + b
        u = jnp.log2(1.0 + jnp.exp2(z * _LOG2E))
        u = jnp.where(z > 60.0, z * _LOG2E, u)
        acc = acc + u
    o_ref[...] = acc * _LOG2 - jnp.float32(N_PERM * _LOG2)


def _activate(x1, b2):
    grid = (N_SITES // _BLK_C,)
    return pl.pallas_call(
        _act_body,
        grid=grid,
        in_specs=[
            pl.BlockSpec((_BLK_C, N_PERM * OUT_FEAT), lambda i: (i, 0)),
            pl.BlockSpec((1, OUT_FEAT), lambda i: (0, 0)),
        ],
        out_specs=pl.BlockSpec((_BLK_C, OUT_FEAT), lambda i: (i, 0)),
        out_shape=jax.ShapeDtypeStruct((N_SITES, OUT_FEAT), jnp.float32),
    )(x1, b2)


def kernel(X_sites, X_NSs, W, b):
    w_big = W.reshape(OUT_FEAT, NBR, D_FEAT).transpose(1, 0, 2).reshape(PROJ, D_FEAT)
    w3 = jnp.pad(w_big, ((0, 2 * D_FEAT - PROJ), (0, 0))).reshape(2, D_FEAT, D_FEAT)

    y = _project(X_sites, w3)
    table = y.reshape(TAB_ROWS, OUT_FEAT)

    nbr_t = jnp.transpose(X_NSs.astype(jnp.int32), (2, 1, 0))
    nbr_t = jnp.pad(nbr_t, ((0, 0), (0, 0), (0, N_PAD - N_SITES)))

    x1 = _gather_sum(table, nbr_t)

    return _activate(x1.reshape(N_PAD, N_PERM * OUT_FEAT), b.reshape(1, OUT_FEAT))
